# Initial kernel scaffold; baseline (speedup 1.0000x reference)
import jax
import jax.numpy as jnp
from jax.experimental import pallas as pl


def kernel(x, Maps, W, b, edge_index):
    raise NotImplementedError("write your pallas kernel here")


# speedup vs baseline: 46.7608x; 46.7608x over previous
import functools

import jax
import jax.numpy as jnp
from jax import lax
from jax.experimental import pallas as pl
from jax.experimental.pallas import tpu as pltpu
from jax.experimental.pallas import tpu_sc as plsc

N = 10000
P = 80000
E = 2 * P
F = 256
H = 128
B = 128
NB = P // B
NW = 32
L = 16


def _splat(val, dtype=jnp.int32):
    return jnp.full((L,), val, dtype=dtype)


def _sc_body(x_hbm, lo_hbm, hi_hbm, wb_hbm, out_hbm,
             lo_v, hi_v, rows_a, rows_b, out_a, out_b, wb_v, sem):
    cid = lax.axis_index("c")
    sid = lax.axis_index("s")
    wid = sid * 2 + cid

    lane = jnp.arange(L, dtype=jnp.int32)

    pltpu.sync_copy(wb_hbm, wb_v)
    wv = [[plsc.load_gather(wb_v, [_splat(8 + j * 4 + i)]) for i in range(4)]
          for j in range(4)]
    bv = [plsc.load_gather(wb_v, [_splat(8 + 16 + j)]) for j in range(4)]

    n_w = (NB - wid + NW - 1) // NW

    def batch_body(t, carry):
        bi = wid + t * NW
        pltpu.sync_copy(lo_hbm.at[pl.ds(bi * B, B)], lo_v)
        pltpu.sync_copy(hi_hbm.at[pl.ds(bi * B, B)], hi_v)
        da = pltpu.async_copy(x_hbm.at[lo_v], rows_a, sem)
        db = pltpu.async_copy(x_hbm.at[hi_v], rows_b, sem)
        da.wait()
        db.wait()

        for g in range(B // L):
            rows = g * L + lane
            zero = jnp.zeros((L,), jnp.float32)

            def h_body(h, acc):
                (c00, c01, c10, c11,
                 ga00, ga01, ga11, gb00, gb01, gb11) = acc
                col = (h + lane) & (H - 1)
                col2 = col + H
                va0 = plsc.load_gather(rows_a, [rows, col])
                va1 = plsc.load_gather(rows_a, [rows, col2])
                vb0 = plsc.load_gather(rows_b, [rows, col])
                vb1 = plsc.load_gather(rows_b, [rows, col2])
                return (c00 + va0 * vb0, c01 + va0 * vb1,
                        c10 + va1 * vb0, c11 + va1 * vb1,
                        ga00 + va0 * va0, ga01 + va0 * va1,
                        ga11 + va1 * va1, gb00 + vb0 * vb0,
                        gb01 + vb0 * vb1, gb11 + vb1 * vb1)

            (c00, c01, c10, c11,
             ga00, ga01, ga11, gb00, gb01, gb11) = lax.fori_loop(
                 0, H, h_body, (zero,) * 10)

            sk = (2.0 * (ga00 - c00), 2.0 * (ga01 - c10),
                  2.0 * (ga01 - c01), 2.0 * (ga11 - c11))
            sp = (2.0 * (gb00 - c00), 2.0 * (gb01 - c01),
                  2.0 * (gb01 - c10), 2.0 * (gb11 - c11))

            oidx = g * (L * 4) + lane * 4
            for j in range(4):
                ya = wv[j][0] * sk[0] + wv[j][1] * sk[1] \
                    + wv[j][2] * sk[2] + wv[j][3] * sk[3] + bv[j]
                yb = wv[j][0] * sp[0] + wv[j][1] * sp[1] \
                    + wv[j][2] * sp[2] + wv[j][3] * sp[3] + bv[j]
                ta = 1.0 - 2.0 / (jnp.exp(jnp.minimum(2.0 * ya, 60.0)) + 1.0)
                tb = 1.0 - 2.0 / (jnp.exp(jnp.minimum(2.0 * yb, 60.0)) + 1.0)
                plsc.store_scatter(out_a, [oidx + j], ta)
                plsc.store_scatter(out_b, [oidx + j], tb)

        pltpu.sync_copy(out_a, out_hbm.at[pl.ds(bi * (B * 4), B * 4)])
        pltpu.sync_copy(out_b, out_hbm.at[pl.ds(P * 4 + bi * (B * 4), B * 4)])
        return carry

    lax.fori_loop(0, n_w, batch_body, 0)


@jax.jit
def _sc_call(x, lo, hi, wb):
    mesh = plsc.VectorSubcoreMesh(core_axis_name="c", subcore_axis_name="s")
    run = pl.kernel(
        _sc_body,
        out_type=jax.ShapeDtypeStruct((E * 4,), jnp.float32),
        mesh=mesh,
        compiler_params=pltpu.CompilerParams(needs_layout_passes=False),
        scratch_types=[
            pltpu.VMEM((B,), jnp.int32),
            pltpu.VMEM((B,), jnp.int32),
            pltpu.VMEM((B, F), jnp.float32),
            pltpu.VMEM((B, F), jnp.float32),
            pltpu.VMEM((B * 4,), jnp.float32),
            pltpu.VMEM((B * 4,), jnp.float32),
            pltpu.VMEM((40,), jnp.float32),
            pltpu.SemaphoreType.DMA,
        ],
    )
    return run(x, lo, hi, wb)


def kernel(x, Maps, W, b, edge_index):
    del Maps
    lo = edge_index[0, :P].astype(jnp.int32)
    hi = edge_index[1, :P].astype(jnp.int32)
    wb = jnp.zeros((40,), jnp.float32)
    wb = wb.at[8:24].set(W.reshape(16).astype(jnp.float32))
    wb = wb.at[24:28].set(b.astype(jnp.float32))
    out = _sc_call(x.astype(jnp.float32), lo, hi, wb)
    return out.reshape(E, 2, 2)


# speedup vs baseline: 48.1861x; 1.0305x over previous
import functools

import jax
import jax.numpy as jnp
from jax import lax
from jax.experimental import pallas as pl
from jax.experimental.pallas import tpu as pltpu
from jax.experimental.pallas import tpu_sc as plsc

N = 10000
P = 80000
E = 2 * P
F = 256
H = 128
B = 128
NB = P // B
NW = 32
L = 16


def _splat(val, dtype=jnp.int32):
    return jnp.full((L,), val, dtype=dtype)


def _gram_body(x_ref, xr_ref, g_ref):
    xr = x_ref[...].astype(jnp.bfloat16).astype(jnp.float32)
    xr_ref[...] = xr
    a0 = xr[:, :H]
    a1 = xr[:, H:]
    g_ref[...] = jnp.concatenate(
        [jnp.sum(a0 * a0, axis=1, keepdims=True),
         jnp.sum(a0 * a1, axis=1, keepdims=True),
         jnp.sum(a1 * a1, axis=1, keepdims=True),
         jnp.zeros((xr.shape[0], 1), jnp.float32)], axis=1)


def _sc_body(x_hbm, g_hbm, lo_hbm, hi_hbm, wb_hbm, out_hbm,
             lo_v, hi_v, rows_a, rows_b, gram_v, out_a, out_b, wb_v, sem):
    cid = lax.axis_index("c")
    sid = lax.axis_index("s")
    wid = sid * 2 + cid

    lane = jnp.arange(L, dtype=jnp.int32)

    pltpu.sync_copy(wb_hbm, wb_v)
    wv = [[plsc.load_gather(wb_v, [_splat(8 + j * 4 + i)]) for i in range(4)]
          for j in range(4)]
    bv = [plsc.load_gather(wb_v, [_splat(8 + 16 + j)]) for j in range(4)]

    pltpu.sync_copy(g_hbm, gram_v)

    n_w = (NB - wid + NW - 1) // NW

    def batch_body(t, carry):
        bi = wid + t * NW
        pltpu.sync_copy(lo_hbm.at[pl.ds(bi * B, B)], lo_v)
        pltpu.sync_copy(hi_hbm.at[pl.ds(bi * B, B)], hi_v)
        da = pltpu.async_copy(x_hbm.at[lo_v], rows_a, sem)
        db = pltpu.async_copy(x_hbm.at[hi_v], rows_b, sem)
        da.wait()
        db.wait()

        for g in range(B // L):
            rows = g * L + lane
            zero = jnp.zeros((L,), jnp.float32)

            def h_body(h4, acc):
                c00, c01, c10, c11 = acc
                for s in range(4):
                    col = (h4 * 4 + s + lane) & (H - 1)
                    col2 = col + H
                    va0 = plsc.load_gather(rows_a, [rows, col])
                    va1 = plsc.load_gather(rows_a, [rows, col2])
                    vb0 = plsc.load_gather(rows_b, [rows, col])
                    vb1 = plsc.load_gather(rows_b, [rows, col2])
                    c00 = c00 + va0 * vb0
                    c01 = c01 + va0 * vb1
                    c10 = c10 + va1 * vb0
                    c11 = c11 + va1 * vb1
                return (c00, c01, c10, c11)

            c00, c01, c10, c11 = lax.fori_loop(
                0, H // 4, h_body, (zero,) * 4)

            la4 = lo_v[pl.ds(g * L, L)] * 4
            lb4 = hi_v[pl.ds(g * L, L)] * 4
            ga00 = plsc.load_gather(gram_v, [la4])
            ga01 = plsc.load_gather(gram_v, [la4 + 1])
            ga11 = plsc.load_gather(gram_v, [la4 + 2])
            gb00 = plsc.load_gather(gram_v, [lb4])
            gb01 = plsc.load_gather(gram_v, [lb4 + 1])
            gb11 = plsc.load_gather(gram_v, [lb4 + 2])

            sk = (2.0 * (ga00 - c00), 2.0 * (ga01 - c10),
                  2.0 * (ga01 - c01), 2.0 * (ga11 - c11))
            sp = (2.0 * (gb00 - c00), 2.0 * (gb01 - c01),
                  2.0 * (gb01 - c10), 2.0 * (gb11 - c11))

            oidx = g * (L * 4) + lane * 4
            for j in range(4):
                ya = wv[j][0] * sk[0] + wv[j][1] * sk[1] \
                    + wv[j][2] * sk[2] + wv[j][3] * sk[3] + bv[j]
                yb = wv[j][0] * sp[0] + wv[j][1] * sp[1] \
                    + wv[j][2] * sp[2] + wv[j][3] * sp[3] + bv[j]
                ta = 1.0 - 2.0 / (jnp.exp(jnp.minimum(2.0 * ya, 60.0)) + 1.0)
                tb = 1.0 - 2.0 / (jnp.exp(jnp.minimum(2.0 * yb, 60.0)) + 1.0)
                plsc.store_scatter(out_a, [oidx + j], ta)
                plsc.store_scatter(out_b, [oidx + j], tb)

        pltpu.sync_copy(out_a, out_hbm.at[pl.ds(bi * (B * 4), B * 4)])
        pltpu.sync_copy(out_b, out_hbm.at[pl.ds(P * 4 + bi * (B * 4), B * 4)])
        return carry

    lax.fori_loop(0, n_w, batch_body, 0)


@jax.jit
def _run(x, lo, hi, wb):
    xr, grams = pl.pallas_call(
        _gram_body,
        out_shape=(jax.ShapeDtypeStruct((N, F), jnp.float32),
                   jax.ShapeDtypeStruct((N, 4), jnp.float32)),
    )(x)
    grams = grams.reshape(N * 4)
    mesh = plsc.VectorSubcoreMesh(core_axis_name="c", subcore_axis_name="s")
    run = pl.kernel(
        _sc_body,
        out_type=jax.ShapeDtypeStruct((E * 4,), jnp.float32),
        mesh=mesh,
        compiler_params=pltpu.CompilerParams(needs_layout_passes=False),
        scratch_types=[
            pltpu.VMEM((B,), jnp.int32),
            pltpu.VMEM((B,), jnp.int32),
            pltpu.VMEM((B, F), jnp.float32),
            pltpu.VMEM((B, F), jnp.float32),
            pltpu.VMEM((N * 4,), jnp.float32),
            pltpu.VMEM((B * 4,), jnp.float32),
            pltpu.VMEM((B * 4,), jnp.float32),
            pltpu.VMEM((40,), jnp.float32),
            pltpu.SemaphoreType.DMA,
        ],
    )
    return run(xr, grams, lo, hi, wb)


def kernel(x, Maps, W, b, edge_index):
    del Maps
    lo = edge_index[0, :P].astype(jnp.int32)
    hi = edge_index[1, :P].astype(jnp.int32)
    wb = jnp.zeros((40,), jnp.float32)
    wb = wb.at[8:24].set(W.reshape(16).astype(jnp.float32))
    wb = wb.at[24:28].set(b.astype(jnp.float32))
    out = _run(x.astype(jnp.float32), lo, hi, wb)
    return out.reshape(E, 2, 2)
